# baseline (speedup 1.0000x reference)
import jax
import jax.numpy as jnp
from jax.experimental import pallas as pl
from jax.experimental.pallas import tpu as pltpu

T = 2048
D = 1024
E = 8
K = 2
I = 512
ISH = 1024
RSF = 2.5

NEG = -1e30
BF = jnp.bfloat16
F32 = jnp.float32

NT = (((1,), (1,)), ((), ()))


def _moe_body(x_ref, gwt_ref, bias_ref, sg_ref, su_ref, sd_ref,
              eg_ref, eu_ref, ed_ref, out_ref, xb_ref, comb_ref):
    e = pl.program_id(0)

    @pl.when(e == 0)
    def _():
        xb_ref[...] = x_ref[...].astype(BF)
        logits = jnp.dot(x_ref[...], gwt_ref[...],
                         preferred_element_type=F32)[:, :E]
        scores = jax.nn.sigmoid(logits)
        sc = scores + bias_ref[...]

        def top2sum(g):
            s = None
            for i in range(4):
                for j in range(i + 1, 4):
                    p = g[:, i:i + 1] + g[:, j:j + 1]
                    s = p if s is None else jnp.maximum(s, p)
            return s

        gs0 = top2sum(sc[:, 0:4])
        gs1 = top2sum(sc[:, 4:8])
        chosen = jnp.where(gs0 >= gs1, 0, 1)
        lane = jax.lax.broadcasted_iota(jnp.int32, (T, E), 1)
        emask = (lane // 4) == chosen
        masked = jnp.where(emask, sc, NEG)
        m1 = jnp.max(masked, axis=1, keepdims=True)
        i1 = jnp.min(jnp.where(masked == m1, lane, E), axis=1, keepdims=True)
        masked2 = jnp.where(lane == i1, NEG, masked)
        m2 = jnp.max(masked2, axis=1, keepdims=True)
        i2 = jnp.min(jnp.where(masked2 == m2, lane, E), axis=1, keepdims=True)
        selmask = jnp.logical_or(lane == i1, lane == i2)
        wsel = jnp.where(selmask, scores, 0.0)
        wsum = jnp.sum(wsel, axis=1, keepdims=True) + 1e-20
        comb = wsel * (RSF / wsum)
        lane16 = jax.lax.broadcasted_iota(jnp.int32, (T, 16), 1)
        shared_cols = jnp.logical_and(lane16 >= E, lane16 < E + 2)
        comb_ref[...] = jnp.where(
            shared_cols, 1.0,
            jnp.where(lane16 < E, jnp.pad(comb, ((0, 0), (0, 8))), 0.0))

    c = jnp.where(e < 2, e + 8, e - 2)
    hot = (jax.lax.broadcasted_iota(jnp.int32, (16, 128), 0) == c
           ).astype(F32)
    col = jax.lax.dot_general(comb_ref[...], hot, (((1,), (0,)), ((), ())),
                              preferred_element_type=F32)[:, :1]

    def mlp(g_w, u_w, d_w):
        xb = xb_ref[...]
        g = jax.lax.dot_general(xb, g_w.astype(BF), NT,
                                preferred_element_type=F32)
        u = jax.lax.dot_general(xb, u_w.astype(BF), NT,
                                preferred_element_type=F32)
        h = (jax.nn.silu(g) * u * col).astype(BF)
        dwb = d_w.astype(BF)
        H = T // 2
        for ci in range(2):
            rows = slice(ci * H, (ci + 1) * H)
            y = jax.lax.dot_general(h[rows, :], dwb, NT,
                                    preferred_element_type=F32)

            @pl.when(e == 0)
            def _():
                out_ref[rows, :] = y

            @pl.when(e > 0)
            def _():
                out_ref[rows, :] = out_ref[rows, :] + y

    @pl.when(e < 2)
    def _():
        mlp(sg_ref[...], su_ref[...], sd_ref[...])

    @pl.when(e >= 2)
    def _():
        mlp(eg_ref[0], eu_ref[0], ed_ref[0])


def kernel(x, max_num_tokens_per_gpu, gate_w, e_score_correction_bias,
           w_shared_gate_up, w_shared_down, w_expert_gate_up, w_expert_down):
    gwt = jnp.zeros((D, 128), F32).at[:, :E].set(gate_w.T)
    bias2 = e_score_correction_bias.reshape(1, E)
    sh = lambda e: (jnp.minimum(e, 1), 0)
    su = lambda e: (2 + jnp.minimum(e, 1), 0)
    sd = lambda e: (0, jnp.minimum(e, 1))
    ex = lambda e: jnp.maximum(e - 2, 0)
    return pl.pallas_call(
        _moe_body,
        grid=(E + 2,),
        in_specs=[
            pl.BlockSpec((T, D), lambda e: (0, 0)),
            pl.BlockSpec((D, 128), lambda e: (0, 0)),
            pl.BlockSpec((1, E), lambda e: (0, 0)),
            pl.BlockSpec((I, D), sh),
            pl.BlockSpec((I, D), su),
            pl.BlockSpec((D, I), sd),
            pl.BlockSpec((1, I, D), lambda e: (ex(e), 0, 0)),
            pl.BlockSpec((1, I, D), lambda e: (ex(e), 1, 0)),
            pl.BlockSpec((1, D, I), lambda e: (ex(e), 0, 0)),
        ],
        out_specs=pl.BlockSpec((T, D), lambda e: (0, 0)),
        out_shape=jax.ShapeDtypeStruct((T, D), F32),
        scratch_shapes=[
            pltpu.VMEM((T, D), BF),
            pltpu.VMEM((T, 16), F32),
        ],
    )(x, gwt, bias2, w_shared_gate_up, w_shared_gate_up, w_shared_down,
      w_expert_gate_up, w_expert_gate_up, w_expert_down)


# speedup vs baseline: 1.0218x; 1.0218x over previous
import jax
import jax.numpy as jnp
from jax.experimental import pallas as pl
from jax.experimental.pallas import tpu as pltpu

T = 2048
D = 1024
E = 8
K = 2
I = 512
ISH = 1024
RSF = 2.5

NEG = -1e30
BF = jnp.bfloat16
F32 = jnp.float32

NT = (((1,), (1,)), ((), ()))


def _moe_body(x_ref, gwt_ref, bias_ref, sg_ref, su_ref, sd_ref,
              eg_ref, eu_ref, ed_ref, out_ref, xb_ref, comb_ref):
    e = pl.program_id(0)

    @pl.when(e == 0)
    def _():
        xb_ref[...] = x_ref[...].astype(BF)
        logits = jnp.dot(x_ref[...], gwt_ref[...],
                         preferred_element_type=F32)[:, :E]
        scores = jax.nn.sigmoid(logits)
        sc = scores + bias_ref[...]

        def top2sum(g):
            s = None
            for i in range(4):
                for j in range(i + 1, 4):
                    p = g[:, i:i + 1] + g[:, j:j + 1]
                    s = p if s is None else jnp.maximum(s, p)
            return s

        gs0 = top2sum(sc[:, 0:4])
        gs1 = top2sum(sc[:, 4:8])
        chosen = jnp.where(gs0 >= gs1, 0, 1)
        lane = jax.lax.broadcasted_iota(jnp.int32, (T, E), 1)
        emask = (lane // 4) == chosen
        masked = jnp.where(emask, sc, NEG)
        m1 = jnp.max(masked, axis=1, keepdims=True)
        i1 = jnp.min(jnp.where(masked == m1, lane, E), axis=1, keepdims=True)
        masked2 = jnp.where(lane == i1, NEG, masked)
        m2 = jnp.max(masked2, axis=1, keepdims=True)
        i2 = jnp.min(jnp.where(masked2 == m2, lane, E), axis=1, keepdims=True)
        selmask = jnp.logical_or(lane == i1, lane == i2)
        wsel = jnp.where(selmask, scores, 0.0)
        wsum = jnp.sum(wsel, axis=1, keepdims=True) + 1e-20
        comb = wsel * (RSF / wsum)
        lane16 = jax.lax.broadcasted_iota(jnp.int32, (T, 16), 1)
        shared_cols = jnp.logical_and(lane16 >= E, lane16 < E + 2)
        comb_ref[...] = jnp.where(
            shared_cols, 1.0,
            jnp.where(lane16 < E, jnp.pad(comb, ((0, 0), (0, 8))), 0.0))

    c = jnp.where(e < 2, e + 8, e - 2)
    hot = (jax.lax.broadcasted_iota(jnp.int32, (16, 128), 0) == c
           ).astype(F32)
    col = jax.lax.dot_general(comb_ref[...], hot, (((1,), (0,)), ((), ())),
                              preferred_element_type=F32)[:, :1]

    def mlp(g_w, u_w, d_w):
        xb = xb_ref[...]
        g = jax.lax.dot_general(xb, g_w.astype(BF), NT,
                                preferred_element_type=F32)
        u = jax.lax.dot_general(xb, u_w.astype(BF), NT,
                                preferred_element_type=F32)
        h = jax.nn.silu(g) * u * col
        y = jax.lax.dot_general(h.astype(BF), d_w.astype(BF), NT,
                                preferred_element_type=F32)

        @pl.when(e == 0)
        def _():
            out_ref[...] = y

        @pl.when(e > 0)
        def _():
            out_ref[...] = out_ref[...] + y

    @pl.when(e < 2)
    def _():
        mlp(sg_ref[...], su_ref[...], sd_ref[...])

    @pl.when(e >= 2)
    def _():
        mlp(eg_ref[0], eu_ref[0], ed_ref[0])


def kernel(x, max_num_tokens_per_gpu, gate_w, e_score_correction_bias,
           w_shared_gate_up, w_shared_down, w_expert_gate_up, w_expert_down):
    gwt = jnp.zeros((D, 128), F32).at[:, :E].set(gate_w.T)
    bias2 = e_score_correction_bias.reshape(1, E)
    sh = lambda e: (jnp.minimum(e, 1), 0)
    su = lambda e: (2 + jnp.minimum(e, 1), 0)
    sd = lambda e: (0, jnp.minimum(e, 1))
    ex = lambda e: jnp.maximum(e - 2, 0)
    return pl.pallas_call(
        _moe_body,
        grid=(E + 2,),
        in_specs=[
            pl.BlockSpec((T, D), lambda e: (0, 0)),
            pl.BlockSpec((D, 128), lambda e: (0, 0)),
            pl.BlockSpec((1, E), lambda e: (0, 0)),
            pl.BlockSpec((I, D), sh),
            pl.BlockSpec((I, D), su),
            pl.BlockSpec((D, I), sd),
            pl.BlockSpec((1, I, D), lambda e: (ex(e), 0, 0)),
            pl.BlockSpec((1, I, D), lambda e: (ex(e), 1, 0)),
            pl.BlockSpec((1, D, I), lambda e: (ex(e), 0, 0)),
        ],
        out_specs=pl.BlockSpec((T, D), lambda e: (0, 0)),
        out_shape=jax.ShapeDtypeStruct((T, D), F32),
        scratch_shapes=[
            pltpu.VMEM((T, D), BF),
            pltpu.VMEM((T, 16), F32),
        ],
    )(x, gwt, bias2, w_shared_gate_up, w_shared_gate_up, w_shared_down,
      w_expert_gate_up, w_expert_gate_up, w_expert_down)
